# Initial kernel scaffold; baseline (speedup 1.0000x reference)
import jax
import jax.numpy as jnp
from jax.experimental import pallas as pl


def kernel(input_values, mask_time_indices, W, codebook):
    raise NotImplementedError("write your pallas kernel here")


# speedup vs baseline: 1.8851x; 1.8851x over previous
import jax
import jax.numpy as jnp
from jax.experimental import pallas as pl


def _vq_body(x_ref, w_ref, cb_ref, out_ref):
    x = x_ref[...]
    w = w_ref[...]
    cb = cb_ref[...]
    t = jax.lax.dot_general(x, w, (((1,), (1,)), ((), ())),
                            preferred_element_type=jnp.float32)
    cross = jax.lax.dot_general(t, cb, (((1,), (0,)), ((), ())),
                                preferred_element_type=jnp.float32)
    c_sq = jnp.sum(cb * cb, axis=0)
    d2 = c_sq[None, :] - 2.0 * cross
    m = jnp.min(d2, axis=1, keepdims=True)
    k = d2.shape[1]
    iota = jax.lax.broadcasted_iota(jnp.int32, d2.shape, 1)
    idx = jnp.min(jnp.where(d2 <= m, iota, k), axis=1)
    out_ref[0, 0, :] = idx


def kernel(input_values, mask_time_indices, W, codebook):
    del mask_time_indices
    b, l, d = input_values.shape
    q, k = codebook.shape
    n = b * l
    blk = 256
    x = input_values.reshape(n, d)
    out = pl.pallas_call(
        _vq_body,
        grid=(n // blk,),
        in_specs=[
            pl.BlockSpec((blk, d), lambda i: (i, 0)),
            pl.BlockSpec((q, d), lambda i: (0, 0)),
            pl.BlockSpec((q, k), lambda i: (0, 0)),
        ],
        out_specs=pl.BlockSpec((1, 1, blk), lambda i: (i, 0, 0)),
        out_shape=jax.ShapeDtypeStruct((n // blk, 1, blk), jnp.int32),
    )(x, W, codebook)
    return out.reshape(b, l)


# speedup vs baseline: 2.0063x; 1.0643x over previous
import jax
import jax.numpy as jnp
from jax.experimental import pallas as pl
from jax.experimental.pallas import tpu as pltpu


def _vq_body(x_ref, w_ref, cb_ref, out_ref, iota_ref):
    q, k = cb_ref.shape
    blk = x_ref.shape[0]

    @pl.when(pl.program_id(0) == 0)
    def _init():
        ii = jax.lax.broadcasted_iota(jnp.int32, (8, k), 1)
        iota_ref[...] = ii.astype(jnp.float32)

    cb = cb_ref[...]
    hc = 0.5 * jnp.sum(cb * cb, axis=0, keepdims=True)
    x = x_ref[...]
    t = jax.lax.dot_general(x, w_ref[...], (((1,), (1,)), ((), ())),
                            preferred_element_type=jnp.float32)
    cross = jax.lax.dot_general(t, cb, (((1,), (0,)), ((), ())),
                                preferred_element_type=jnp.float32)
    adj = cross - hc
    m = jnp.max(adj, axis=1, keepdims=True)
    iota_f = iota_ref[0:1, :]
    sel = jnp.where(adj >= m, iota_f, jnp.float32(k))
    out_ref[0, 0, :] = jnp.min(sel, axis=1).astype(jnp.int32)


def kernel(input_values, mask_time_indices, W, codebook):
    del mask_time_indices
    b, l, d = input_values.shape
    q, k = codebook.shape
    n = b * l
    blk = 256
    x = input_values.reshape(n, d)
    out = pl.pallas_call(
        _vq_body,
        grid=(n // blk,),
        in_specs=[
            pl.BlockSpec((blk, d), lambda i: (i, 0)),
            pl.BlockSpec((q, d), lambda i: (0, 0)),
            pl.BlockSpec((q, k), lambda i: (0, 0)),
        ],
        out_specs=pl.BlockSpec((1, 1, blk), lambda i: (i, 0, 0)),
        out_shape=jax.ShapeDtypeStruct((n // blk, 1, blk), jnp.int32),
        scratch_shapes=[pltpu.VMEM((8, k), jnp.float32)],
    )(x, W, codebook)
    return out.reshape(b, l)
